# baseline (speedup 1.0000x reference)
import functools

import jax
import jax.numpy as jnp
from jax import lax
from jax.experimental import pallas as pl
from jax.experimental.pallas import tpu as pltpu
from jax.experimental.pallas import tpu_sc as plsc


def _router_body(T, E, N, x_ref, w1t_ref, b1_ref, w2t_ref, b2_ref,
                 i0_ref, i1_ref, g0_ref, g1_ref):
    i = pl.program_id(0)
    h = jnp.dot(x_ref[...], w1t_ref[...], preferred_element_type=jnp.float32)
    h = jnp.maximum(h + b1_ref[...], 0.0)
    logits = jnp.dot(h, w2t_ref[...], preferred_element_type=jnp.float32)
    logits = logits + b2_ref[...]
    m = jnp.max(logits, axis=1, keepdims=True)
    p = jnp.exp(logits - m)
    p = p / jnp.sum(p, axis=1, keepdims=True)
    lane = lax.broadcasted_iota(jnp.int32, (T, E), 1)
    p1 = jnp.max(p, axis=1, keepdims=True)
    i1 = jnp.min(jnp.where(p == p1, lane, E), axis=1, keepdims=True)
    pm = jnp.where(lane == i1, -jnp.inf, p)
    p2 = jnp.max(pm, axis=1, keepdims=True)
    i2 = jnp.min(jnp.where(pm == p2, lane, E), axis=1, keepdims=True)
    tok = i * T + lax.broadcasted_iota(jnp.int32, (T, 1), 0)
    i0_ref[...] = jnp.broadcast_to(i1 * N + tok, (T, 128))
    i1_ref[...] = jnp.broadcast_to(i2 * N + tok, (T, 128))
    g0_ref[...] = jnp.broadcast_to(p1, (T, 128))
    g1_ref[...] = jnp.broadcast_to(p2, (T, 128))


def _router(x, w1t, b1, w2t, b2, T=512):
    N, H = x.shape
    E = w2t.shape[1]
    body = functools.partial(_router_body, T, E, N)
    grid = (N // T,)
    outs = pl.pallas_call(
        body,
        grid=grid,
        in_specs=[
            pl.BlockSpec((T, H), lambda i: (i, 0)),
            pl.BlockSpec((H, H), lambda i: (0, 0)),
            pl.BlockSpec((1, H), lambda i: (0, 0)),
            pl.BlockSpec((H, E), lambda i: (0, 0)),
            pl.BlockSpec((1, E), lambda i: (0, 0)),
        ],
        out_specs=[
            pl.BlockSpec((T, 128), lambda i: (i, 0)),
            pl.BlockSpec((T, 128), lambda i: (i, 0)),
            pl.BlockSpec((T, 128), lambda i: (i, 0)),
            pl.BlockSpec((T, 128), lambda i: (i, 0)),
        ],
        out_shape=[
            jax.ShapeDtypeStruct((N, 128), jnp.int32),
            jax.ShapeDtypeStruct((N, 128), jnp.int32),
            jax.ShapeDtypeStruct((N, 128), jnp.float32),
            jax.ShapeDtypeStruct((N, 128), jnp.float32),
        ],
    )(x, w1t, b1, w2t, b2)
    return outs


def _make_combine(N, H, G=32):
    n_workers = 32
    per_w = N // n_workers
    n_chunks = per_w // G
    mesh = plsc.VectorSubcoreMesh(
        core_axis_name="c", subcore_axis_name="s", num_cores=2, num_subcores=16)

    @functools.partial(
        pl.kernel,
        out_type=jax.ShapeDtypeStruct((N, H), jnp.float32),
        mesh=mesh,
        scratch_types=[
            pltpu.VMEM((G,), jnp.int32),
            pltpu.VMEM((G,), jnp.int32),
            pltpu.VMEM((G, 128), jnp.float32),
            pltpu.VMEM((G, 128), jnp.float32),
            pltpu.VMEM((G, H), jnp.float32),
            pltpu.VMEM((G, H), jnp.float32),
            pltpu.VMEM((G, H), jnp.float32),
            pltpu.SemaphoreType.DMA,
        ],
    )
    def combine(table, i0, i1, g0, g1, out,
                i0_v, i1_v, g0_v, g1_v, r0_v, r1_v, o_v, sem):
        wid = lax.axis_index("s") * 2 + lax.axis_index("c")

        @pl.loop(0, n_chunks)
        def _chunk(c):
            base = wid * per_w + c * G
            pltpu.sync_copy(i0.at[pl.ds(base, G)], i0_v)
            pltpu.sync_copy(i1.at[pl.ds(base, G)], i1_v)
            pltpu.sync_copy(g0.at[pl.ds(base, G)], g0_v)
            pltpu.sync_copy(g1.at[pl.ds(base, G)], g1_v)
            pltpu.async_copy(table.at[i0_v], r0_v, sem).wait()
            pltpu.async_copy(table.at[i1_v], r1_v, sem).wait()

            @pl.loop(0, G)
            def _tok(t):
                ga = g0_v[t, pl.ds(0, 16)]
                gb = g1_v[t, pl.ds(0, 16)]
                for j in range(H // 16):
                    sl = pl.ds(j * 16, 16)
                    o_v[t, sl] = r0_v[t, sl] * ga + r1_v[t, sl] * gb

            pltpu.sync_copy(o_v, out.at[pl.ds(base, G)])

    return combine


def kernel(hidden_states, expert_outputs, W1, b1, W2, b2):
    B, S, H = hidden_states.shape
    E = W2.shape[0]
    N = B * S
    x = hidden_states.reshape(N, H)
    table = expert_outputs.reshape(E * N, H)
    i0r, i1r, g0r, g1r = _router(
        x, W1.T, b1.reshape(1, H), W2.T, b2.reshape(1, E))
    i0 = i0r[:, 0]
    i1 = i1r[:, 0]
    out = _make_combine(N, H)(table, i0, i1, g0r, g1r)
    return out.reshape(B, S, H)


# speedup vs baseline: 1.2493x; 1.2493x over previous
import functools

import jax
import jax.numpy as jnp
from jax import lax
from jax.experimental import pallas as pl
from jax.experimental.pallas import tpu as pltpu
from jax.experimental.pallas import tpu_sc as plsc


def _router_body(T, E, N, x_ref, w1t_ref, b1_ref, w2t_ref, b2_ref,
                 i0_ref, i1_ref, g0_ref, g1_ref):
    i = pl.program_id(0)
    h = jnp.dot(x_ref[...], w1t_ref[...], preferred_element_type=jnp.float32)
    h = jnp.maximum(h + b1_ref[...], 0.0)
    logits = jnp.dot(h, w2t_ref[...], preferred_element_type=jnp.float32)
    logits = logits + b2_ref[...]
    m = jnp.max(logits, axis=1, keepdims=True)
    p = jnp.exp(logits - m)
    p = p / jnp.sum(p, axis=1, keepdims=True)
    lane = lax.broadcasted_iota(jnp.int32, (T, E), 1)
    p1 = jnp.max(p, axis=1, keepdims=True)
    i1 = jnp.min(jnp.where(p == p1, lane, E), axis=1, keepdims=True)
    pm = jnp.where(lane == i1, -jnp.inf, p)
    p2 = jnp.max(pm, axis=1, keepdims=True)
    i2 = jnp.min(jnp.where(pm == p2, lane, E), axis=1, keepdims=True)
    tok = i * T + lax.broadcasted_iota(jnp.int32, (T, 1), 0)
    i0_ref[...] = jnp.broadcast_to(i1 * N + tok, (T, 128))
    i1_ref[...] = jnp.broadcast_to(i2 * N + tok, (T, 128))
    g0_ref[...] = jnp.broadcast_to(p1, (T, 128))
    g1_ref[...] = jnp.broadcast_to(p2, (T, 128))


def _router(x, w1t, b1, w2t, b2, T=512):
    N, H = x.shape
    E = w2t.shape[1]
    body = functools.partial(_router_body, T, E, N)
    grid = (N // T,)
    outs = pl.pallas_call(
        body,
        grid=grid,
        in_specs=[
            pl.BlockSpec((T, H), lambda i: (i, 0)),
            pl.BlockSpec((H, H), lambda i: (0, 0)),
            pl.BlockSpec((1, H), lambda i: (0, 0)),
            pl.BlockSpec((H, E), lambda i: (0, 0)),
            pl.BlockSpec((1, E), lambda i: (0, 0)),
        ],
        out_specs=[
            pl.BlockSpec((T, 128), lambda i: (i, 0)),
            pl.BlockSpec((T, 128), lambda i: (i, 0)),
            pl.BlockSpec((T, 128), lambda i: (i, 0)),
            pl.BlockSpec((T, 128), lambda i: (i, 0)),
        ],
        out_shape=[
            jax.ShapeDtypeStruct((N, 128), jnp.int32),
            jax.ShapeDtypeStruct((N, 128), jnp.int32),
            jax.ShapeDtypeStruct((N, 128), jnp.float32),
            jax.ShapeDtypeStruct((N, 128), jnp.float32),
        ],
    )(x, w1t, b1, w2t, b2)
    return outs


def _make_combine(N, H, G=8):
    n_workers = 32
    per_w = N // n_workers
    n_chunks = per_w // G
    assert n_chunks % 2 == 0
    mesh = plsc.VectorSubcoreMesh(
        core_axis_name="c", subcore_axis_name="s", num_cores=2, num_subcores=16)

    @functools.partial(
        pl.kernel,
        out_type=jax.ShapeDtypeStruct((N, H), jnp.float32),
        mesh=mesh,
        scratch_types=[
            pltpu.VMEM((2 * per_w,), jnp.int32),
            pltpu.VMEM((2 * per_w * 16,), jnp.float32),
            pltpu.VMEM((2, 2 * G, H), jnp.float32),
            pltpu.VMEM((2, G, H), jnp.float32),
            pltpu.SemaphoreType.DMA,
            pltpu.SemaphoreType.DMA,
            pltpu.SemaphoreType.DMA,
            pltpu.SemaphoreType.DMA,
        ],
    )
    def combine(table, i0, i1, g0, g1, out,
                idx_v, g_v, r_v, o_v, sg0, sg1, so0, so1):
        wid = lax.axis_index("s") * 2 + lax.axis_index("c")
        wbase = wid * per_w
        sg = (sg0, sg1)
        so = (so0, so1)

        pltpu.sync_copy(i0.at[pl.ds(wbase, per_w)], idx_v.at[pl.ds(0, per_w)])
        pltpu.sync_copy(i1.at[pl.ds(wbase, per_w)],
                        idx_v.at[pl.ds(per_w, per_w)])
        pltpu.sync_copy(g0.at[pl.ds(wbase * 16, per_w * 16)],
                        g_v.at[pl.ds(0, per_w * 16)])
        pltpu.sync_copy(g1.at[pl.ds(wbase * 16, per_w * 16)],
                        g_v.at[pl.ds(per_w * 16, per_w * 16)])

        def gather_descs(cc, b):
            base = cc * G
            d0 = pltpu.make_async_copy(
                table.at[idx_v.at[pl.ds(base, G)]],
                r_v.at[b, pl.ds(0, G)], sg[b])
            d1 = pltpu.make_async_copy(
                table.at[idx_v.at[pl.ds(per_w + base, G)]],
                r_v.at[b, pl.ds(G, G)], sg[b])
            return d0, d1

        def out_desc(cc, b):
            return pltpu.make_async_copy(
                o_v.at[b], out.at[pl.ds(wbase + cc * G, G)], so[b])

        d0, d1 = gather_descs(0, 0)
        d0.start()
        d1.start()

        @pl.loop(0, n_chunks, step=2)
        def _pair(c):
            for b in range(2):
                cc = c + b
                w0, w1 = gather_descs(cc, b)
                w0.wait()
                w1.wait()

                @pl.when(cc + 1 < n_chunks)
                def _():
                    n0, n1 = gather_descs(cc + 1, 1 - b)
                    n0.start()
                    n1.start()

                @pl.when(cc >= 2)
                def _():
                    out_desc(cc, b).wait()

                @pl.loop(0, G)
                def _tok(t):
                    ga = g_v[pl.ds((cc * G + t) * 16, 16)]
                    gb = g_v[pl.ds((per_w + cc * G + t) * 16, 16)]
                    for j in range(H // 16):
                        sl = pl.ds(j * 16, 16)
                        o_v[b, t, sl] = (r_v[b, t, sl] * ga
                                         + r_v[b, G + t, sl] * gb)

                out_desc(cc, b).start()

        out_desc(n_chunks - 2, 0).wait()
        out_desc(n_chunks - 1, 1).wait()

    return combine


def kernel(hidden_states, expert_outputs, W1, b1, W2, b2):
    B, S, H = hidden_states.shape
    E = W2.shape[0]
    N = B * S
    x = hidden_states.reshape(N, H)
    table = expert_outputs.reshape(E * N, H)
    i0r, i1r, g0r, g1r = _router(
        x, W1.T, b1.reshape(1, H), W2.T, b2.reshape(1, E))
    i0 = i0r[:, 0]
    i1 = i1r[:, 0]
    g0 = g0r[:, :16].reshape(N * 16)
    g1 = g1r[:, :16].reshape(N * 16)
    out = _make_combine(N, H)(table, i0, i1, g0, g1)
    return out.reshape(B, S, H)


# speedup vs baseline: 1.3057x; 1.0451x over previous
import functools

import jax
import jax.numpy as jnp
from jax import lax
from jax.experimental import pallas as pl
from jax.experimental.pallas import tpu as pltpu
from jax.experimental.pallas import tpu_sc as plsc


def _router_body(T, E, N, x_ref, w1t_ref, b1_ref, w2t_ref, b2_ref,
                 i0_ref, i1_ref, g0_ref, g1_ref):
    i = pl.program_id(0)
    h = jnp.dot(x_ref[...], w1t_ref[...], preferred_element_type=jnp.float32)
    h = jnp.maximum(h + b1_ref[...], 0.0)
    logits = jnp.dot(h, w2t_ref[...], preferred_element_type=jnp.float32)
    logits = logits + b2_ref[...]
    m = jnp.max(logits, axis=1, keepdims=True)
    p = jnp.exp(logits - m)
    p = p / jnp.sum(p, axis=1, keepdims=True)
    lane = lax.broadcasted_iota(jnp.int32, (T, E), 1)
    p1 = jnp.max(p, axis=1, keepdims=True)
    i1 = jnp.min(jnp.where(p == p1, lane, E), axis=1, keepdims=True)
    pm = jnp.where(lane == i1, -jnp.inf, p)
    p2 = jnp.max(pm, axis=1, keepdims=True)
    i2 = jnp.min(jnp.where(pm == p2, lane, E), axis=1, keepdims=True)
    tok = i * T + lax.broadcasted_iota(jnp.int32, (T, 1), 0)
    i0_ref[...] = jnp.broadcast_to(i1 * N + tok, (T, 16))
    i1_ref[...] = jnp.broadcast_to(i2 * N + tok, (T, 16))
    g0_ref[...] = jnp.broadcast_to(p1, (T, 16))
    g1_ref[...] = jnp.broadcast_to(p2, (T, 16))


def _router(x, w1t, b1, w2t, b2, T=1024):
    N, H = x.shape
    E = w2t.shape[1]
    body = functools.partial(_router_body, T, E, N)
    grid = (N // T,)
    outs = pl.pallas_call(
        body,
        grid=grid,
        in_specs=[
            pl.BlockSpec((T, H), lambda i: (i, 0)),
            pl.BlockSpec((H, H), lambda i: (0, 0)),
            pl.BlockSpec((1, H), lambda i: (0, 0)),
            pl.BlockSpec((H, E), lambda i: (0, 0)),
            pl.BlockSpec((1, E), lambda i: (0, 0)),
        ],
        out_specs=[
            pl.BlockSpec((T, 16), lambda i: (i, 0)),
            pl.BlockSpec((T, 16), lambda i: (i, 0)),
            pl.BlockSpec((T, 16), lambda i: (i, 0)),
            pl.BlockSpec((T, 16), lambda i: (i, 0)),
        ],
        out_shape=[
            jax.ShapeDtypeStruct((N, 16), jnp.int32),
            jax.ShapeDtypeStruct((N, 16), jnp.int32),
            jax.ShapeDtypeStruct((N, 16), jnp.float32),
            jax.ShapeDtypeStruct((N, 16), jnp.float32),
        ],
    )(x, w1t, b1, w2t, b2)
    return outs


def _make_combine(N, H, G=8):
    n_workers = 32
    per_w = N // n_workers
    n_chunks = per_w // G
    assert n_chunks % 2 == 0
    mesh = plsc.VectorSubcoreMesh(
        core_axis_name="c", subcore_axis_name="s", num_cores=2, num_subcores=16)

    @functools.partial(
        pl.kernel,
        out_type=jax.ShapeDtypeStruct((N, H), jnp.float32),
        mesh=mesh,
        scratch_types=[
            pltpu.VMEM((2 * per_w,), jnp.int32),
            pltpu.VMEM((2 * per_w * 16,), jnp.float32),
            pltpu.VMEM((2, 2 * G, H), jnp.float32),
            pltpu.VMEM((2, G, H), jnp.float32),
            pltpu.SemaphoreType.DMA,
            pltpu.SemaphoreType.DMA,
            pltpu.SemaphoreType.DMA,
            pltpu.SemaphoreType.DMA,
        ],
    )
    def combine(table, i0, i1, g0, g1, out,
                idx_v, g_v, r_v, o_v, sg0, sg1, so0, so1):
        wid = lax.axis_index("s") * 2 + lax.axis_index("c")
        wbase = wid * per_w
        sg = (sg0, sg1)
        so = (so0, so1)

        pltpu.sync_copy(i0.at[pl.ds(wbase, per_w)], idx_v.at[pl.ds(0, per_w)])
        pltpu.sync_copy(i1.at[pl.ds(wbase, per_w)],
                        idx_v.at[pl.ds(per_w, per_w)])
        pltpu.sync_copy(g0.at[pl.ds(wbase * 16, per_w * 16)],
                        g_v.at[pl.ds(0, per_w * 16)])
        pltpu.sync_copy(g1.at[pl.ds(wbase * 16, per_w * 16)],
                        g_v.at[pl.ds(per_w * 16, per_w * 16)])

        def gather_descs(cc, b):
            base = cc * G
            d0 = pltpu.make_async_copy(
                table.at[idx_v.at[pl.ds(base, G)]],
                r_v.at[b, pl.ds(0, G)], sg[b])
            d1 = pltpu.make_async_copy(
                table.at[idx_v.at[pl.ds(per_w + base, G)]],
                r_v.at[b, pl.ds(G, G)], sg[b])
            return d0, d1

        def out_desc(cc, b):
            return pltpu.make_async_copy(
                o_v.at[b], out.at[pl.ds(wbase + cc * G, G)], so[b])

        d0, d1 = gather_descs(0, 0)
        d0.start()
        d1.start()

        @pl.loop(0, n_chunks, step=2)
        def _pair(c):
            for b in range(2):
                cc = c + b
                w0, w1 = gather_descs(cc, b)
                w0.wait()
                w1.wait()

                @pl.when(cc + 1 < n_chunks)
                def _():
                    n0, n1 = gather_descs(cc + 1, 1 - b)
                    n0.start()
                    n1.start()

                @pl.when(cc >= 2)
                def _():
                    out_desc(cc, b).wait()

                @pl.loop(0, G)
                def _tok(t):
                    ga = g_v[pl.ds((cc * G + t) * 16, 16)]
                    gb = g_v[pl.ds((per_w + cc * G + t) * 16, 16)]
                    for j in range(H // 16):
                        sl = pl.ds(j * 16, 16)
                        o_v[b, t, sl] = (r_v[b, t, sl] * ga
                                         + r_v[b, G + t, sl] * gb)

                out_desc(cc, b).start()

        out_desc(n_chunks - 2, 0).wait()
        out_desc(n_chunks - 1, 1).wait()

    return combine


def kernel(hidden_states, expert_outputs, W1, b1, W2, b2):
    B, S, H = hidden_states.shape
    E = W2.shape[0]
    N = B * S
    x = hidden_states.reshape(N, H)
    table = expert_outputs.reshape(E * N, H)
    i0r, i1r, g0r, g1r = _router(
        x, W1.T, b1.reshape(1, H), W2.T, b2.reshape(1, E))
    i0 = i0r[:, 0]
    i1 = i1r[:, 0]
    g0 = g0r.reshape(N * 16)
    g1 = g1r.reshape(N * 16)
    out = _make_combine(N, H)(table, i0, i1, g0, g1)
    return out.reshape(B, S, H)


# speedup vs baseline: 1.4502x; 1.1107x over previous
import functools

import jax
import jax.numpy as jnp
from jax import lax
from jax.experimental import pallas as pl
from jax.experimental.pallas import tpu as pltpu
from jax.experimental.pallas import tpu_sc as plsc


def _router_body(T, E, N, x_ref, w1t_ref, b1_ref, w2t_ref, b2_ref,
                 i0_ref, i1_ref, g0_ref, g1_ref):
    i = pl.program_id(0)
    h = jnp.dot(x_ref[...], w1t_ref[...], preferred_element_type=jnp.float32)
    h = jnp.maximum(h + b1_ref[...], 0.0)
    logits = jnp.dot(h, w2t_ref[...], preferred_element_type=jnp.float32)
    logits = logits + b2_ref[...]
    m = jnp.max(logits, axis=1, keepdims=True)
    p = jnp.exp(logits - m)
    p = p / jnp.sum(p, axis=1, keepdims=True)
    lane = lax.broadcasted_iota(jnp.int32, (T, E), 1)
    p1 = jnp.max(p, axis=1, keepdims=True)
    i1 = jnp.min(jnp.where(p == p1, lane, E), axis=1, keepdims=True)
    pm = jnp.where(lane == i1, -jnp.inf, p)
    p2 = jnp.max(pm, axis=1, keepdims=True)
    i2 = jnp.min(jnp.where(pm == p2, lane, E), axis=1, keepdims=True)
    tok = i * T + lax.broadcasted_iota(jnp.int32, (T, 1), 0)
    i0_ref[...] = jnp.broadcast_to(i1 * N + tok, (T, 16))
    i1_ref[...] = jnp.broadcast_to(i2 * N + tok, (T, 16))
    g0_ref[...] = jnp.broadcast_to(p1, (T, 16))
    g1_ref[...] = jnp.broadcast_to(p2, (T, 16))


def _router(x, w1t, b1, w2t, b2, T=1024):
    N, H = x.shape
    E = w2t.shape[1]
    body = functools.partial(_router_body, T, E, N)
    grid = (N // T,)
    outs = pl.pallas_call(
        body,
        grid=grid,
        in_specs=[
            pl.BlockSpec((T, H), lambda i: (i, 0)),
            pl.BlockSpec((H, H), lambda i: (0, 0)),
            pl.BlockSpec((1, H), lambda i: (0, 0)),
            pl.BlockSpec((H, E), lambda i: (0, 0)),
            pl.BlockSpec((1, E), lambda i: (0, 0)),
        ],
        out_specs=[
            pl.BlockSpec((T, 16), lambda i: (i, 0)),
            pl.BlockSpec((T, 16), lambda i: (i, 0)),
            pl.BlockSpec((T, 16), lambda i: (i, 0)),
            pl.BlockSpec((T, 16), lambda i: (i, 0)),
        ],
        out_shape=[
            jax.ShapeDtypeStruct((N, 16), jnp.int32),
            jax.ShapeDtypeStruct((N, 16), jnp.int32),
            jax.ShapeDtypeStruct((N, 16), jnp.float32),
            jax.ShapeDtypeStruct((N, 16), jnp.float32),
        ],
    )(x, w1t, b1, w2t, b2)
    return outs


def _make_combine(N, H, G=8):
    n_workers = 32
    per_w = N // n_workers
    n_chunks = per_w // G
    assert n_chunks % 4 == 0
    mesh = plsc.VectorSubcoreMesh(
        core_axis_name="c", subcore_axis_name="s", num_cores=2, num_subcores=16)

    @functools.partial(
        pl.kernel,
        out_type=jax.ShapeDtypeStruct((N, H), jnp.float32),
        mesh=mesh,
        scratch_types=[
            pltpu.VMEM((2 * per_w,), jnp.int32),
            pltpu.VMEM((2 * per_w * 16,), jnp.float32),
            pltpu.VMEM((4, 2 * G, H), jnp.float32),
            pltpu.VMEM((4, G, H), jnp.float32),
            pltpu.SemaphoreType.DMA,
            pltpu.SemaphoreType.DMA,
            pltpu.SemaphoreType.DMA,
            pltpu.SemaphoreType.DMA,
            pltpu.SemaphoreType.DMA,
            pltpu.SemaphoreType.DMA,
            pltpu.SemaphoreType.DMA,
            pltpu.SemaphoreType.DMA,
        ],
    )
    def combine(table, i0, i1, g0, g1, out,
                idx_v, g_v, r_v, o_v,
                sg0, sg1, sg2, sg3, so0, so1, so2, so3):
        wid = lax.axis_index("s") * 2 + lax.axis_index("c")
        wbase = wid * per_w
        sg = (sg0, sg1, sg2, sg3)
        so = (so0, so1, so2, so3)

        pltpu.sync_copy(i0.at[pl.ds(wbase, per_w)], idx_v.at[pl.ds(0, per_w)])
        pltpu.sync_copy(i1.at[pl.ds(wbase, per_w)],
                        idx_v.at[pl.ds(per_w, per_w)])
        pltpu.sync_copy(g0.at[pl.ds(wbase * 16, per_w * 16)],
                        g_v.at[pl.ds(0, per_w * 16)])
        pltpu.sync_copy(g1.at[pl.ds(wbase * 16, per_w * 16)],
                        g_v.at[pl.ds(per_w * 16, per_w * 16)])

        def gather_descs(cc, b):
            base = cc * G
            d0 = pltpu.make_async_copy(
                table.at[idx_v.at[pl.ds(base, G)]],
                r_v.at[b, pl.ds(0, G)], sg[b])
            d1 = pltpu.make_async_copy(
                table.at[idx_v.at[pl.ds(per_w + base, G)]],
                r_v.at[b, pl.ds(G, G)], sg[b])
            return d0, d1

        def out_desc(cc, b):
            return pltpu.make_async_copy(
                o_v.at[b], out.at[pl.ds(wbase + cc * G, G)], so[b])

        for pre in range(3):
            d0, d1 = gather_descs(pre, pre)
            d0.start()
            d1.start()

        @pl.loop(0, n_chunks, step=4)
        def _quad(c):
            for b in range(4):
                cc = c + b
                w0, w1 = gather_descs(cc, b)
                w0.wait()
                w1.wait()

                @pl.when(cc + 3 < n_chunks)
                def _():
                    n0, n1 = gather_descs(cc + 3, (b + 3) % 4)
                    n0.start()
                    n1.start()

                @pl.when(cc >= 4)
                def _():
                    out_desc(cc, b).wait()

                @pl.loop(0, G)
                def _tok(t):
                    ga = g_v[pl.ds((cc * G + t) * 16, 16)]
                    gb = g_v[pl.ds((per_w + cc * G + t) * 16, 16)]
                    for j in range(H // 16):
                        sl = pl.ds(j * 16, 16)
                        o_v[b, t, sl] = (r_v[b, t, sl] * ga
                                         + r_v[b, G + t, sl] * gb)

                out_desc(cc, b).start()

        for tail in range(4):
            out_desc(n_chunks - 4 + tail, tail).wait()

    return combine


def kernel(hidden_states, expert_outputs, W1, b1, W2, b2):
    B, S, H = hidden_states.shape
    E = W2.shape[0]
    N = B * S
    x = hidden_states.reshape(N, H)
    table = expert_outputs.reshape(E * N, H)
    i0r, i1r, g0r, g1r = _router(
        x, W1.T, b1.reshape(1, H), W2.T, b2.reshape(1, E))
    i0 = i0r[:, 0]
    i1 = i1r[:, 0]
    g0 = g0r.reshape(N * 16)
    g1 = g1r.reshape(N * 16)
    out = _make_combine(N, H)(table, i0, i1, g0, g1)
    return out.reshape(B, S, H)
